# baseline (speedup 1.0000x reference)
import jax
import jax.numpy as jnp
from jax import lax
from jax.experimental import pallas as pl
from jax.experimental.pallas import tpu as pltpu
from jax.experimental.pallas import tpu_sc as plsc

NUM_KV_HEADS = 8
KV_HEAD_DIM = 64
KV_DIM = NUM_KV_HEADS * KV_HEAD_DIM
LANE = 128

_info = plsc.get_sparse_core_info()
NC = _info.num_cores
NS = _info.num_subcores
NW = NC * NS

GSUB = 32
TBLOCK = 128
N_LAYERS = 3
RSPLIT = KV_DIM // LANE


def _build_vrow_indices(ids_ref, gidx_ref):
    n = gidx_ref.shape[0]
    iota = lax.iota(jnp.int32, 16)

    @plsc.parallel_loop(0, n // 16, unroll=2)
    def _(v):
        p = iota + v * 16
        ids16 = plsc.load_gather(ids_ref, [p // RSPLIT])
        ct = jnp.bitwise_and(p, RSPLIT - 1)
        row = (
            (ids16 >> 3) * (8 * RSPLIT)
            + ct * 8
            + jnp.bitwise_and(ids16, 7)
        )
        gidx_ref[pl.ds(v * 16, 16)] = row


def _transpose_sub(rows_ref, trans_ref, col0):
    iota = lax.iota(jnp.int32, 16)
    j8 = iota // 8
    jm8 = iota % 8
    rot = [jnp.bitwise_and(iota + k, 15) for k in range(16)]
    ldc = [r * RSPLIT for r in rot]
    stc = [r + col0 for r in rot]
    nhalf = GSUB // 16
    nqg = KV_DIM // 16

    @plsc.parallel_loop(0, nhalf * nqg, unroll=1)
    def _(u):
        thalf = u % nhalf
        qg = u // nhalf
        q0 = qg * 16
        cvec = iota + (q0 % LANE)
        i0 = j8 + (q0 // 8)
        rbase = q0 // LANE + thalf * (16 * RSPLIT)
        t0v = jnp.full((16,), thalf * 16, dtype=jnp.int32)
        for k in range(16):
            g = plsc.load_gather(rows_ref, [ldc[k] + rbase, cvec])
            plsc.store_scatter(trans_ref, [i0, jm8, stc[k] + t0v], g)


def _ve_body(
    ids_hbm, w0_hbm, w1_hbm, w2_hbm, out_hbm,
    idx_v, gidx_v, rows0, rows1, trans_v, sem_in0, sem_in1, sem_out,
):
    n_ids = ids_hbm.shape[0]
    n_tile_cols = out_hbm.shape[3]
    rows_per_w = n_ids // NW
    w_per_b = (n_tile_cols * TBLOCK) // rows_per_w
    wid = lax.axis_index("s") * NC + lax.axis_index("c")
    b_idx = wid // w_per_b
    tc0 = (wid % w_per_b) * (rows_per_w // TBLOCK)
    pltpu.sync_copy(ids_hbm.at[pl.ds(wid * rows_per_w, rows_per_w)], idx_v)
    _build_vrow_indices(idx_v, gidx_v)

    rows = (rows0, rows1)
    sem_in = (sem_in0, sem_in1)
    tables = (w0_hbm, w1_hbm, w2_hbm)

    blocks_per_l = rows_per_w // TBLOCK
    subs_per_block = TBLOCK // GSUB
    idx_per_sub = GSUB * RSPLIT

    def gather_copy(table, s, buf):
        return pltpu.make_async_copy(
            table.at[gidx_v.at[pl.ds(s * idx_per_sub, idx_per_sub)]],
            rows[buf],
            sem_in[buf],
        )

    def write_copy(l, k):
        return pltpu.make_async_copy(
            trans_v,
            out_hbm.at[l, b_idx, :, tc0 + k, :, :],
            sem_out,
        )

    gather_copy(tables[0], 0, 0).start()
    for l, table in enumerate(tables):

        def blk_body(k, _, table=table, l=l):
            s0 = k * subs_per_block
            for j in range(subs_per_block):
                gather_copy(table, s0 + j, j % 2).wait()
                if j < subs_per_block - 1:
                    gather_copy(table, s0 + j + 1, (j + 1) % 2).start()
                else:
                    @pl.when(k < blocks_per_l - 1)
                    def _():
                        gather_copy(table, s0 + j + 1, (j + 1) % 2).start()
                if j == 0:
                    @pl.when(k > 0)
                    def _():
                        write_copy(l, k - 1).wait()
                _transpose_sub(rows[j % 2], trans_v, j * GSUB)
            write_copy(l, k).start()
            return 0

        lax.fori_loop(0, blocks_per_l, blk_body, 0)
        if l < N_LAYERS - 1:
            gather_copy(tables[l + 1], 0, 0).start()
        write_copy(l, blocks_per_l - 1).wait()


def _virtual_rows(w):
    v = w.shape[0]
    w4 = w.reshape(v // 8, 8, RSPLIT, LANE)
    return jnp.transpose(w4, (0, 2, 1, 3)).reshape(v * RSPLIT, LANE)


@jax.jit
def kernel(input_ids, w0, w1, w2):
    b, t = input_ids.shape
    n_ids = b * t
    rows_per_w = n_ids // NW
    ids_flat = input_ids.reshape(n_ids)
    mesh = plsc.VectorSubcoreMesh(core_axis_name="c", subcore_axis_name="s")
    out6 = pl.kernel(
        _ve_body,
        out_type=jax.ShapeDtypeStruct(
            (N_LAYERS, b, KV_DIM // 8, t // TBLOCK, 8, TBLOCK), jnp.float32
        ),
        mesh=mesh,
        scratch_types=[
            pltpu.VMEM((rows_per_w,), jnp.int32),
            pltpu.VMEM((rows_per_w * RSPLIT,), jnp.int32),
            pltpu.VMEM((GSUB * RSPLIT, LANE), jnp.float32),
            pltpu.VMEM((GSUB * RSPLIT, LANE), jnp.float32),
            pltpu.VMEM((KV_DIM // 8, 8, TBLOCK), jnp.float32),
            pltpu.SemaphoreType.DMA,
            pltpu.SemaphoreType.DMA,
            pltpu.SemaphoreType.DMA,
        ],
        compiler_params=pltpu.CompilerParams(
            use_tc_tiling_on_sc=False, needs_layout_passes=False
        ),
    )(ids_flat, _virtual_rows(w0), _virtual_rows(w1), _virtual_rows(w2))
    out7 = out6.reshape(N_LAYERS, b, NUM_KV_HEADS, 8, t // TBLOCK, 8, TBLOCK)
    y = jnp.transpose(out7, (0, 1, 4, 6, 2, 3, 5))
    return y.reshape(N_LAYERS, b, t, NUM_KV_HEADS, KV_HEAD_DIM)


# speedup vs baseline: 1.2911x; 1.2911x over previous
import jax
import jax.numpy as jnp
from jax import lax
from jax.experimental import pallas as pl
from jax.experimental.pallas import tpu as pltpu
from jax.experimental.pallas import tpu_sc as plsc

NUM_KV_HEADS = 8
KV_HEAD_DIM = 64
KV_DIM = NUM_KV_HEADS * KV_HEAD_DIM
LANE = 128

_info = plsc.get_sparse_core_info()
NC = _info.num_cores
NS = _info.num_subcores
NW = NC * NS

TBLOCK = 128
N_LAYERS = 3
RSPLIT = KV_DIM // LANE


def _build_vrow_indices(ids_ref, gidx_ref):
    n = gidx_ref.shape[0]
    iota = lax.iota(jnp.int32, 16)

    @plsc.parallel_loop(0, n // 16, unroll=2)
    def _(v):
        p = iota + v * 16
        tidx = ((p >> 9) << 7) + jnp.bitwise_and(p, LANE - 1)
        ids16 = plsc.load_gather(ids_ref, [tidx])
        ct = jnp.bitwise_and(p >> 7, RSPLIT - 1)
        row = (
            (ids16 >> 3) * (8 * RSPLIT)
            + ct * 8
            + jnp.bitwise_and(ids16, 7)
        )
        gidx_ref[pl.ds(v * 16, 16)] = row


def _transpose_sub(rows_ref, trans_ref):
    iota = lax.iota(jnp.int32, 16)
    j8 = iota // 8
    jm8 = iota % 8
    rot = [jnp.bitwise_and(iota + k, 15) for k in range(16)]
    ngrp = TBLOCK // 16

    @plsc.parallel_loop(0, ngrp * ngrp, unroll=1)
    def _(u):
        tg = u % ngrp
        cg = u // ngrp
        c0 = cg * 16
        cvec = iota + c0
        i0 = j8 + (cg * 2)
        t0v = jnp.full((16,), tg * 16, dtype=jnp.int32)
        for k in range(16):
            g = plsc.load_gather(rows_ref, [rot[k] + t0v, cvec])
            plsc.store_scatter(trans_ref, [i0, jm8, rot[k] + t0v], g)


def _ve_body(
    ids_hbm, w0_hbm, w1_hbm, w2_hbm, out_hbm,
    idx_v, gidx_v, rows0, rows1, trans0, trans1,
    sem_in0, sem_in1, sem_out0, sem_out1,
):
    n_ids = ids_hbm.shape[0]
    n_tile_cols = out_hbm.shape[3]
    rows_per_w = n_ids // NW
    w_per_b = (n_tile_cols * TBLOCK) // rows_per_w
    wid = lax.axis_index("s") * NC + lax.axis_index("c")
    b_idx = wid // w_per_b
    tc0 = (wid % w_per_b) * (rows_per_w // TBLOCK)
    pltpu.sync_copy(ids_hbm.at[pl.ds(wid * rows_per_w, rows_per_w)], idx_v)
    _build_vrow_indices(idx_v, gidx_v)

    rows = (rows0, rows1)
    trans = (trans0, trans1)
    sem_in = (sem_in0, sem_in1)
    sem_out = (sem_out0, sem_out1)
    tables = (w0_hbm, w1_hbm, w2_hbm)

    blocks_per_l = rows_per_w // TBLOCK

    def gather_copy(table, k, ct):
        return pltpu.make_async_copy(
            table.at[gidx_v.at[pl.ds((k * RSPLIT + ct) * TBLOCK, TBLOCK)]],
            rows[ct % 2],
            sem_in[ct % 2],
        )

    def write_copy(l, k, ct):
        return pltpu.make_async_copy(
            trans[ct % 2],
            out_hbm.at[l, b_idx, pl.ds(ct * 16, 16), tc0 + k, :, :],
            sem_out[ct % 2],
        )

    gather_copy(tables[0], 0, 0).start()
    for l, table in enumerate(tables):

        def blk_body(k, _, table=table, l=l):
            for ct in range(RSPLIT):
                gather_copy(table, k, ct).wait()
                if ct < RSPLIT - 1:
                    gather_copy(table, k, ct + 1).start()
                else:
                    @pl.when(k < blocks_per_l - 1)
                    def _():
                        gather_copy(table, k + 1, 0).start()
                if ct < 2:
                    if l == 0:
                        @pl.when(k > 0)
                        def _():
                            write_copy(l, k, ct).wait()
                    else:
                        write_copy(l, k, ct).wait()
                else:
                    write_copy(l, k, ct).wait()
                _transpose_sub(rows[ct % 2], trans[ct % 2])
                write_copy(l, k, ct).start()
            return 0

        lax.fori_loop(0, blocks_per_l, blk_body, 0)
        if l < N_LAYERS - 1:
            gather_copy(tables[l + 1], 0, 0).start()
    write_copy(N_LAYERS - 1, blocks_per_l - 1, RSPLIT - 2).wait()
    write_copy(N_LAYERS - 1, blocks_per_l - 1, RSPLIT - 1).wait()


def _virtual_rows(w):
    v = w.shape[0]
    w4 = w.reshape(v // 8, 8, RSPLIT, LANE)
    return jnp.transpose(w4, (0, 2, 1, 3)).reshape(v * RSPLIT, LANE)


@jax.jit
def kernel(input_ids, w0, w1, w2):
    b, t = input_ids.shape
    n_ids = b * t
    rows_per_w = n_ids // NW
    ids_flat = input_ids.reshape(n_ids)
    mesh = plsc.VectorSubcoreMesh(core_axis_name="c", subcore_axis_name="s")
    out6 = pl.kernel(
        _ve_body,
        out_type=jax.ShapeDtypeStruct(
            (N_LAYERS, b, KV_DIM // 8, t // TBLOCK, 8, TBLOCK), jnp.float32
        ),
        mesh=mesh,
        scratch_types=[
            pltpu.VMEM((rows_per_w,), jnp.int32),
            pltpu.VMEM((rows_per_w * RSPLIT,), jnp.int32),
            pltpu.VMEM((TBLOCK, LANE), jnp.float32),
            pltpu.VMEM((TBLOCK, LANE), jnp.float32),
            pltpu.VMEM((16, 8, TBLOCK), jnp.float32),
            pltpu.VMEM((16, 8, TBLOCK), jnp.float32),
            pltpu.SemaphoreType.DMA,
            pltpu.SemaphoreType.DMA,
            pltpu.SemaphoreType.DMA,
            pltpu.SemaphoreType.DMA,
        ],
        compiler_params=pltpu.CompilerParams(
            use_tc_tiling_on_sc=False, needs_layout_passes=False
        ),
    )(ids_flat, _virtual_rows(w0), _virtual_rows(w1), _virtual_rows(w2))
    out7 = out6.reshape(N_LAYERS, b, NUM_KV_HEADS, 8, t // TBLOCK, 8, TBLOCK)
    y = jnp.transpose(out7, (0, 1, 4, 6, 2, 3, 5))
    return y.reshape(N_LAYERS, b, t, NUM_KV_HEADS, KV_HEAD_DIM)


# speedup vs baseline: 1.3029x; 1.0091x over previous
import jax
import jax.numpy as jnp
from jax import lax
from jax.experimental import pallas as pl
from jax.experimental.pallas import tpu as pltpu
from jax.experimental.pallas import tpu_sc as plsc

NUM_KV_HEADS = 8
KV_HEAD_DIM = 64
KV_DIM = NUM_KV_HEADS * KV_HEAD_DIM
LANE = 128

_info = plsc.get_sparse_core_info()
NC = _info.num_cores
NS = _info.num_subcores
NW = NC * NS

TBLOCK = 128
N_LAYERS = 3
RSPLIT = KV_DIM // LANE


def _build_vrow_indices(ids_ref, gidx_ref):
    n = gidx_ref.shape[0]
    iota = lax.iota(jnp.int32, 16)

    @plsc.parallel_loop(0, n // 16, unroll=2)
    def _(v):
        p = iota + v * 16
        tidx = ((p >> 9) << 7) + jnp.bitwise_and(p, LANE - 1)
        ids16 = plsc.load_gather(ids_ref, [tidx])
        ct = jnp.bitwise_and(p >> 7, RSPLIT - 1)
        row = (
            (ids16 >> 3) * (8 * RSPLIT)
            + ct * 8
            + jnp.bitwise_and(ids16, 7)
        )
        gidx_ref[pl.ds(v * 16, 16)] = row


def _transpose_sub(rows_ref, trans_ref):
    iota = lax.iota(jnp.int32, 16)
    j8 = iota // 8
    jm8 = iota % 8
    rot = [jnp.bitwise_and(iota + k, 15) for k in range(16)]
    ngrp = TBLOCK // 16

    @plsc.parallel_loop(0, ngrp * ngrp, unroll=2)
    def _(u):
        tg = u % ngrp
        cg = u // ngrp
        c0 = cg * 16
        cvec = iota + c0
        i0 = j8 + (cg * 2)
        t0v = jnp.full((16,), tg * 16, dtype=jnp.int32)
        for k in range(16):
            g = plsc.load_gather(rows_ref, [rot[k] + t0v, cvec])
            plsc.store_scatter(trans_ref, [i0, jm8, rot[k] + t0v], g)


def _ve_body(
    ids_hbm, w0_hbm, w1_hbm, w2_hbm, out_hbm,
    idx_v, gidx_v, rows0, rows1, trans0, trans1,
    sem_in0, sem_in1, sem_out0, sem_out1,
):
    n_ids = ids_hbm.shape[0]
    n_tile_cols = out_hbm.shape[3]
    rows_per_w = n_ids // NW
    w_per_b = (n_tile_cols * TBLOCK) // rows_per_w
    wid = lax.axis_index("s") * NC + lax.axis_index("c")
    b_idx = wid // w_per_b
    tc0 = (wid % w_per_b) * (rows_per_w // TBLOCK)
    pltpu.sync_copy(ids_hbm.at[pl.ds(wid * rows_per_w, rows_per_w)], idx_v)
    _build_vrow_indices(idx_v, gidx_v)

    rows = (rows0, rows1)
    trans = (trans0, trans1)
    sem_in = (sem_in0, sem_in1)
    sem_out = (sem_out0, sem_out1)
    tables = (w0_hbm, w1_hbm, w2_hbm)

    blocks_per_l = rows_per_w // TBLOCK

    def gather_copy(table, k, ct):
        return pltpu.make_async_copy(
            table.at[gidx_v.at[pl.ds((k * RSPLIT + ct) * TBLOCK, TBLOCK)]],
            rows[ct % 2],
            sem_in[ct % 2],
        )

    def write_copy(l, k, ct):
        return pltpu.make_async_copy(
            trans[ct % 2],
            out_hbm.at[l, b_idx, pl.ds(ct * 16, 16), tc0 + k, :, :],
            sem_out[ct % 2],
        )

    gather_copy(tables[0], 0, 0).start()
    for l, table in enumerate(tables):

        def blk_body(k, _, table=table, l=l):
            for ct in range(RSPLIT):
                gather_copy(table, k, ct).wait()
                if ct < RSPLIT - 1:
                    gather_copy(table, k, ct + 1).start()
                else:
                    @pl.when(k < blocks_per_l - 1)
                    def _():
                        gather_copy(table, k + 1, 0).start()
                if ct < 2:
                    if l == 0:
                        @pl.when(k > 0)
                        def _():
                            write_copy(l, k, ct).wait()
                    else:
                        write_copy(l, k, ct).wait()
                else:
                    write_copy(l, k, ct).wait()
                _transpose_sub(rows[ct % 2], trans[ct % 2])
                write_copy(l, k, ct).start()
            return 0

        lax.fori_loop(0, blocks_per_l, blk_body, 0)
        if l < N_LAYERS - 1:
            gather_copy(tables[l + 1], 0, 0).start()
    write_copy(N_LAYERS - 1, blocks_per_l - 1, RSPLIT - 2).wait()
    write_copy(N_LAYERS - 1, blocks_per_l - 1, RSPLIT - 1).wait()


def _virtual_rows(w):
    v = w.shape[0]
    w4 = w.reshape(v // 8, 8, RSPLIT, LANE)
    return jnp.transpose(w4, (0, 2, 1, 3)).reshape(v * RSPLIT, LANE)


@jax.jit
def kernel(input_ids, w0, w1, w2):
    b, t = input_ids.shape
    n_ids = b * t
    rows_per_w = n_ids // NW
    ids_flat = input_ids.reshape(n_ids)
    mesh = plsc.VectorSubcoreMesh(core_axis_name="c", subcore_axis_name="s")
    out6 = pl.kernel(
        _ve_body,
        out_type=jax.ShapeDtypeStruct(
            (N_LAYERS, b, KV_DIM // 8, t // TBLOCK, 8, TBLOCK), jnp.float32
        ),
        mesh=mesh,
        scratch_types=[
            pltpu.VMEM((rows_per_w,), jnp.int32),
            pltpu.VMEM((rows_per_w * RSPLIT,), jnp.int32),
            pltpu.VMEM((TBLOCK, LANE), jnp.float32),
            pltpu.VMEM((TBLOCK, LANE), jnp.float32),
            pltpu.VMEM((16, 8, TBLOCK), jnp.float32),
            pltpu.VMEM((16, 8, TBLOCK), jnp.float32),
            pltpu.SemaphoreType.DMA,
            pltpu.SemaphoreType.DMA,
            pltpu.SemaphoreType.DMA,
            pltpu.SemaphoreType.DMA,
        ],
        compiler_params=pltpu.CompilerParams(
            use_tc_tiling_on_sc=False, needs_layout_passes=False
        ),
    )(ids_flat, _virtual_rows(w0), _virtual_rows(w1), _virtual_rows(w2))
    out7 = out6.reshape(N_LAYERS, b, NUM_KV_HEADS, 8, t // TBLOCK, 8, TBLOCK)
    y = jnp.transpose(out7, (0, 1, 4, 6, 2, 3, 5))
    return y.reshape(N_LAYERS, b, t, NUM_KV_HEADS, KV_HEAD_DIM)


# speedup vs baseline: 1.3032x; 1.0003x over previous
import jax
import jax.numpy as jnp
from jax import lax
from jax.experimental import pallas as pl
from jax.experimental.pallas import tpu as pltpu
from jax.experimental.pallas import tpu_sc as plsc

NUM_KV_HEADS = 8
KV_HEAD_DIM = 64
KV_DIM = NUM_KV_HEADS * KV_HEAD_DIM
LANE = 128

_info = plsc.get_sparse_core_info()
NC = _info.num_cores
NS = _info.num_subcores
NW = NC * NS

TBLOCK = 128
N_LAYERS = 3
RSPLIT = KV_DIM // LANE


def _build_vrow_indices(ids_ref, gidx_ref, lo, hi):
    iota = lax.iota(jnp.int32, 16)

    @plsc.parallel_loop(lo, hi, unroll=2)
    def _(v):
        p = iota + v * 16
        tidx = ((p >> 9) << 7) + jnp.bitwise_and(p, LANE - 1)
        ids16 = plsc.load_gather(ids_ref, [tidx])
        ct = jnp.bitwise_and(p >> 7, RSPLIT - 1)
        row = (
            (ids16 >> 3) * (8 * RSPLIT)
            + ct * 8
            + jnp.bitwise_and(ids16, 7)
        )
        gidx_ref[pl.ds(v * 16, 16)] = row


def _transpose_sub(rows_ref, trans_ref):
    iota = lax.iota(jnp.int32, 16)
    j8 = iota // 8
    jm8 = iota % 8
    rot = [jnp.bitwise_and(iota + k, 15) for k in range(16)]
    ngrp = TBLOCK // 16

    @plsc.parallel_loop(0, ngrp * ngrp, unroll=2)
    def _(u):
        tg = u % ngrp
        cg = u // ngrp
        c0 = cg * 16
        cvec = iota + c0
        i0 = j8 + (cg * 2)
        t0v = jnp.full((16,), tg * 16, dtype=jnp.int32)
        for k in range(16):
            g = plsc.load_gather(rows_ref, [rot[k] + t0v, cvec])
            plsc.store_scatter(trans_ref, [i0, jm8, rot[k] + t0v], g)


def _ve_body(
    ids_hbm, w0_hbm, w1_hbm, w2_hbm, out_hbm,
    idx_v, gidx_v, rows0, rows1, trans0, trans1,
    sem_in0, sem_in1, sem_out0, sem_out1,
):
    n_ids = ids_hbm.shape[0]
    n_tile_cols = out_hbm.shape[3]
    rows_per_w = n_ids // NW
    w_per_b = (n_tile_cols * TBLOCK) // rows_per_w
    wid = lax.axis_index("s") * NC + lax.axis_index("c")
    b_idx = wid // w_per_b
    tc0 = (wid % w_per_b) * (rows_per_w // TBLOCK)
    pltpu.sync_copy(ids_hbm.at[pl.ds(wid * rows_per_w, rows_per_w)], idx_v)
    n_gidx16 = gidx_v.shape[0] // 16
    _build_vrow_indices(idx_v, gidx_v, 0, TBLOCK // 16)

    rows = (rows0, rows1)
    trans = (trans0, trans1)
    sem_in = (sem_in0, sem_in1)
    sem_out = (sem_out0, sem_out1)
    tables = (w0_hbm, w1_hbm, w2_hbm)

    blocks_per_l = rows_per_w // TBLOCK

    def gather_copy(table, k, ct):
        return pltpu.make_async_copy(
            table.at[gidx_v.at[pl.ds((k * RSPLIT + ct) * TBLOCK, TBLOCK)]],
            rows[ct % 2],
            sem_in[ct % 2],
        )

    def write_copy(l, k, ct):
        return pltpu.make_async_copy(
            trans[ct % 2],
            out_hbm.at[l, b_idx, pl.ds(ct * 16, 16), tc0 + k, :, :],
            sem_out[ct % 2],
        )

    gather_copy(tables[0], 0, 0).start()
    _build_vrow_indices(idx_v, gidx_v, TBLOCK // 16, n_gidx16)
    for l, table in enumerate(tables):

        def blk_body(k, _, table=table, l=l):
            for ct in range(RSPLIT):
                gather_copy(table, k, ct).wait()
                if ct < RSPLIT - 1:
                    gather_copy(table, k, ct + 1).start()
                else:
                    @pl.when(k < blocks_per_l - 1)
                    def _():
                        gather_copy(table, k + 1, 0).start()
                if ct < 2:
                    if l == 0:
                        @pl.when(k > 0)
                        def _():
                            write_copy(l, k, ct).wait()
                    else:
                        write_copy(l, k, ct).wait()
                else:
                    write_copy(l, k, ct).wait()
                _transpose_sub(rows[ct % 2], trans[ct % 2])
                write_copy(l, k, ct).start()
            return 0

        lax.fori_loop(0, blocks_per_l, blk_body, 0)
        if l < N_LAYERS - 1:
            gather_copy(tables[l + 1], 0, 0).start()
    write_copy(N_LAYERS - 1, blocks_per_l - 1, RSPLIT - 2).wait()
    write_copy(N_LAYERS - 1, blocks_per_l - 1, RSPLIT - 1).wait()


def _virtual_rows(w):
    v = w.shape[0]
    w4 = w.reshape(v // 8, 8, RSPLIT, LANE)
    return jnp.transpose(w4, (0, 2, 1, 3)).reshape(v * RSPLIT, LANE)


@jax.jit
def kernel(input_ids, w0, w1, w2):
    b, t = input_ids.shape
    n_ids = b * t
    rows_per_w = n_ids // NW
    ids_flat = input_ids.reshape(n_ids)
    mesh = plsc.VectorSubcoreMesh(core_axis_name="c", subcore_axis_name="s")
    out6 = pl.kernel(
        _ve_body,
        out_type=jax.ShapeDtypeStruct(
            (N_LAYERS, b, KV_DIM // 8, t // TBLOCK, 8, TBLOCK), jnp.float32
        ),
        mesh=mesh,
        scratch_types=[
            pltpu.VMEM((rows_per_w,), jnp.int32),
            pltpu.VMEM((rows_per_w * RSPLIT,), jnp.int32),
            pltpu.VMEM((TBLOCK, LANE), jnp.float32),
            pltpu.VMEM((TBLOCK, LANE), jnp.float32),
            pltpu.VMEM((16, 8, TBLOCK), jnp.float32),
            pltpu.VMEM((16, 8, TBLOCK), jnp.float32),
            pltpu.SemaphoreType.DMA,
            pltpu.SemaphoreType.DMA,
            pltpu.SemaphoreType.DMA,
            pltpu.SemaphoreType.DMA,
        ],
        compiler_params=pltpu.CompilerParams(
            use_tc_tiling_on_sc=False, needs_layout_passes=False
        ),
    )(ids_flat, _virtual_rows(w0), _virtual_rows(w1), _virtual_rows(w2))
    out7 = out6.reshape(N_LAYERS, b, NUM_KV_HEADS, 8, t // TBLOCK, 8, TBLOCK)
    y = jnp.transpose(out7, (0, 1, 4, 6, 2, 3, 5))
    return y.reshape(N_LAYERS, b, t, NUM_KV_HEADS, KV_HEAD_DIM)
